# baseline (speedup 1.0000x reference)
import functools

import jax
import jax.numpy as jnp
from jax.experimental import pallas as pl
from jax.experimental.pallas import tpu as pltpu
from jax.experimental.pallas import tpu_sc as plsc

T = 4096
H = 2048
E = 8
O2 = 4096
OC = 4
CH = O2 // OC

NC = 2
NS = 16
NW = NC * NS
TPW = T // NW
CTOK = 16
NCH = TPW // CTOK
LANES = 16
HV = H // LANES

_sc_mesh = plsc.VectorSubcoreMesh(core_axis_name="c", subcore_axis_name="s")


def _sc_body(spR_hbm, hid_hbm, out_hbm, sp_v, buf0, buf1, acc_v, sem0, sem1):
    c = jax.lax.axis_index("c")
    s = jax.lax.axis_index("s")
    wid = s * NC + c
    base = wid * TPW
    pltpu.sync_copy(spR_hbm.at[wid], sp_v)
    bufs = (buf0, buf1)
    sems = (sem0, sem1)
    cps = [pltpu.async_copy(hid_hbm.at[pl.ds(base, CTOK), :], bufs[0], sems[0]),
           None]
    for ci in range(NCH):
        if ci + 1 < NCH:
            nb = (ci + 1) % 2
            cps[nb] = pltpu.async_copy(
                hid_hbm.at[pl.ds(base + (ci + 1) * CTOK, CTOK), :],
                bufs[nb], sems[nb])
        cps[ci % 2].wait()
        buf = bufs[ci % 2]
        spvecs = [sp_v[e, pl.ds(ci * CTOK, CTOK)] for e in range(E)]
        spw = [[spvecs[e][i] for e in range(E)] for i in range(CTOK)]
        first = ci == 0

        @plsc.parallel_loop(0, HV, 1)
        def hv_body(hv, buf=buf, spw=spw, first=first):
            col = pl.ds(hv * LANES, LANES)
            if first:
                acc_a = [jnp.zeros((LANES,), jnp.float32) for _ in range(E)]
            else:
                acc_a = [acc_v[e, col] for e in range(E)]
            acc_b = [jnp.zeros((LANES,), jnp.float32) for _ in range(E)]
            half = CTOK // 2
            for i in range(half):
                v = buf[i, col]
                for e in range(E):
                    acc_a[e] = acc_a[e] + spw[i][e] * v
            for i in range(half, CTOK):
                v = buf[i, col]
                for e in range(E):
                    acc_b[e] = acc_b[e] + spw[i][e] * v
            for e in range(E):
                acc_v[e, col] = acc_a[e] + acc_b[e]
    pltpu.sync_copy(acc_v, out_hbm.at[wid])


_sc_sh = functools.partial(
    pl.kernel,
    out_type=jax.ShapeDtypeStruct((NW, E, H), jnp.float32),
    mesh=_sc_mesh,
    scratch_types=[
        pltpu.VMEM((E, TPW), jnp.float32),
        pltpu.VMEM((CTOK, H), jnp.float32),
        pltpu.VMEM((CTOK, H), jnp.float32),
        pltpu.VMEM((E, H), jnp.float32),
        pltpu.SemaphoreType.DMA,
        pltpu.SemaphoreType.DMA,
    ],
)(_sc_body)


def _ws_body(w_ref, ws_ref):
    e = pl.program_id(0)
    oc = pl.program_id(1)

    @pl.when((e == 0) & (oc == 0))
    def _init():
        ws_ref[...] = jnp.zeros_like(ws_ref)

    ws_ref[pl.ds(e, 1), :] += jnp.sum(w_ref[0], axis=-1)[None, :]


def _fin_body(shp_ref, ws_ref, out_ref):
    sh = jnp.sum(shp_ref[...], axis=0)
    out_ref[...] = jnp.sum(sh * ws_ref[...]).reshape(1, 1)


def kernel(hidden_4d, sparsity, gate_up_proj):
    hidden = hidden_4d.reshape(T, H)
    spR = sparsity.reshape(E, NW, TPW).transpose(1, 0, 2)
    w = gate_up_proj.reshape(E, H, O2)

    shp = _sc_sh(spR, hidden)

    ws = pl.pallas_call(
        _ws_body,
        grid=(E, OC),
        in_specs=[pl.BlockSpec((1, H, CH), lambda e, oc: (e, 0, oc))],
        out_specs=pl.BlockSpec((E, H), lambda e, oc: (0, 0)),
        out_shape=jax.ShapeDtypeStruct((E, H), jnp.float32),
    )(w)

    out = pl.pallas_call(
        _fin_body,
        in_specs=[pl.BlockSpec((NW, E, H), lambda: (0, 0, 0)),
                  pl.BlockSpec((E, H), lambda: (0, 0))],
        out_specs=pl.BlockSpec((1, 1), lambda: (0, 0)),
        out_shape=jax.ShapeDtypeStruct((1, 1), jnp.float32),
    )(shp, ws)
    return out[0, 0]


# speedup vs baseline: 2.5436x; 2.5436x over previous
import functools

import jax
import jax.numpy as jnp
from jax.experimental import pallas as pl
from jax.experimental.pallas import tpu as pltpu
from jax.experimental.pallas import tpu_sc as plsc

T = 4096
H = 2048
E = 8
O2 = 4096
OC = 4
CH = O2 // OC

NC = 2
NS = 16
NW = NC * NS
TPW = T // NW
CTOK = 8
NCH = TPW // CTOK
LANES = 16
HV = H // LANES

_sc_mesh = plsc.VectorSubcoreMesh(core_axis_name="c", subcore_axis_name="s")


def _sc_body(spR_hbm, hid_hbm, out_hbm, sp_v, buf0, buf1, acc_v, sem0, sem1):
    c = jax.lax.axis_index("c")
    s = jax.lax.axis_index("s")
    wid = s * NC + c
    base = wid * TPW
    pltpu.sync_copy(spR_hbm.at[wid], sp_v)
    bufs = (buf0, buf1)
    sems = (sem0, sem1)
    cps = [pltpu.async_copy(hid_hbm.at[pl.ds(base, CTOK), :], bufs[0], sems[0]),
           None]
    for ci in range(NCH):
        if ci + 1 < NCH:
            nb = (ci + 1) % 2
            cps[nb] = pltpu.async_copy(
                hid_hbm.at[pl.ds(base + (ci + 1) * CTOK, CTOK), :],
                bufs[nb], sems[nb])
        cps[ci % 2].wait()
        buf = bufs[ci % 2]
        lo = (ci % 2) * CTOK
        spvecs = [sp_v[e, pl.ds((ci // 2) * 16, 16)] for e in range(E)]
        spw = [[spvecs[e][lo + i] for e in range(E)] for i in range(CTOK)]
        first = ci == 0

        def hv_body(hv, carry, buf=buf, spw=spw, first=first):
            cb = hv * (2 * LANES)
            cols = (pl.ds(cb, LANES), pl.ds(cb + LANES, LANES))
            for col in cols:
                if first:
                    accs = [jnp.zeros((LANES,), jnp.float32)
                            for _ in range(E)]
                else:
                    accs = [acc_v[e, col] for e in range(E)]
                for i in range(CTOK):
                    v = buf[i, col]
                    for e in range(E):
                        accs[e] = accs[e] + spw[i][e] * v
                for e in range(E):
                    acc_v[e, col] = accs[e]
            return carry

        jax.lax.fori_loop(0, HV // 2, hv_body, 0)
    pltpu.sync_copy(acc_v, out_hbm.at[wid])


_sc_sh = functools.partial(
    pl.kernel,
    out_type=jax.ShapeDtypeStruct((NW, E, H), jnp.float32),
    mesh=_sc_mesh,
    scratch_types=[
        pltpu.VMEM((E, TPW), jnp.float32),
        pltpu.VMEM((CTOK, H), jnp.float32),
        pltpu.VMEM((CTOK, H), jnp.float32),
        pltpu.VMEM((E, H), jnp.float32),
        pltpu.SemaphoreType.DMA,
        pltpu.SemaphoreType.DMA,
    ],
)(_sc_body)


def _ws_body(w_ref, ws_ref):
    e = pl.program_id(0)
    oc = pl.program_id(1)

    @pl.when((e == 0) & (oc == 0))
    def _init():
        ws_ref[...] = jnp.zeros_like(ws_ref)

    ws_ref[pl.ds(e, 1), :] += jnp.sum(w_ref[0], axis=-1)[None, :]


def _fin_body(shp_ref, ws_ref, out_ref):
    sh = jnp.sum(shp_ref[...], axis=0)
    out_ref[...] = jnp.sum(sh * ws_ref[...]).reshape(1, 1)


def kernel(hidden_4d, sparsity, gate_up_proj):
    hidden = hidden_4d.reshape(T, H)
    spR = sparsity.reshape(E, NW, TPW).transpose(1, 0, 2)
    w = gate_up_proj.reshape(E, H, O2)

    shp = _sc_sh(spR, hidden)

    ws = pl.pallas_call(
        _ws_body,
        grid=(E, OC),
        in_specs=[pl.BlockSpec((1, H, CH), lambda e, oc: (e, 0, oc))],
        out_specs=pl.BlockSpec((E, H), lambda e, oc: (0, 0)),
        out_shape=jax.ShapeDtypeStruct((E, H), jnp.float32),
    )(w)

    out = pl.pallas_call(
        _fin_body,
        in_specs=[pl.BlockSpec((NW, E, H), lambda: (0, 0, 0)),
                  pl.BlockSpec((E, H), lambda: (0, 0))],
        out_specs=pl.BlockSpec((1, 1), lambda: (0, 0)),
        out_shape=jax.ShapeDtypeStruct((1, 1), jnp.float32),
    )(shp, ws)
    return out[0, 0]


# speedup vs baseline: 2.6661x; 1.0482x over previous
import functools

import jax
import jax.numpy as jnp
from jax.experimental import pallas as pl
from jax.experimental.pallas import tpu as pltpu
from jax.experimental.pallas import tpu_sc as plsc

T = 4096
H = 2048
E = 8
O2 = 4096
OC = 4
CH = O2 // OC

NC = 2
NS = 16
NW = NC * NS
TPW = T // NW
CTOK = 8
NCH = TPW // CTOK
LANES = 16
HV = H // LANES

_sc_mesh = plsc.VectorSubcoreMesh(core_axis_name="c", subcore_axis_name="s")


def _sc_body(spR_hbm, hid_hbm, out_hbm, sp_v, buf0, buf1, acc_v, sem0, sem1):
    c = jax.lax.axis_index("c")
    s = jax.lax.axis_index("s")
    wid = s * NC + c
    base = wid * TPW
    pltpu.sync_copy(spR_hbm.at[wid], sp_v)
    bufs = (buf0, buf1)
    sems = (sem0, sem1)
    cps = [pltpu.async_copy(hid_hbm.at[pl.ds(base, CTOK), :], bufs[0], sems[0]),
           None]
    for ci in range(NCH):
        if ci + 1 < NCH:
            nb = (ci + 1) % 2
            cps[nb] = pltpu.async_copy(
                hid_hbm.at[pl.ds(base + (ci + 1) * CTOK, CTOK), :],
                bufs[nb], sems[nb])
        cps[ci % 2].wait()
        buf = bufs[ci % 2]
        lo = (ci % 2) * CTOK
        spvecs = [sp_v[e, pl.ds((ci // 2) * 16, 16)] for e in range(E)]
        spw = [[spvecs[e][lo + i] for e in range(E)] for i in range(CTOK)]
        first = ci == 0

        def hv_body(hv, carry, buf=buf, spw=spw, first=first):
            cb = hv * (4 * LANES)
            cols = tuple(pl.ds(cb + j * LANES, LANES) for j in range(4))
            for col in cols:
                if first:
                    accs = [jnp.zeros((LANES,), jnp.float32)
                            for _ in range(E)]
                else:
                    accs = [acc_v[e, col] for e in range(E)]
                for i in range(CTOK):
                    v = buf[i, col]
                    for e in range(E):
                        accs[e] = accs[e] + spw[i][e] * v
                for e in range(E):
                    acc_v[e, col] = accs[e]
            return carry

        jax.lax.fori_loop(0, HV // 4, hv_body, 0)
    pltpu.sync_copy(acc_v, out_hbm.at[wid])


_sc_sh = functools.partial(
    pl.kernel,
    out_type=jax.ShapeDtypeStruct((NW, E, H), jnp.float32),
    mesh=_sc_mesh,
    scratch_types=[
        pltpu.VMEM((E, TPW), jnp.float32),
        pltpu.VMEM((CTOK, H), jnp.float32),
        pltpu.VMEM((CTOK, H), jnp.float32),
        pltpu.VMEM((E, H), jnp.float32),
        pltpu.SemaphoreType.DMA,
        pltpu.SemaphoreType.DMA,
    ],
)(_sc_body)


def _ws_body(w_ref, ws_ref):
    e = pl.program_id(0)
    oc = pl.program_id(1)

    @pl.when((e == 0) & (oc == 0))
    def _init():
        ws_ref[...] = jnp.zeros_like(ws_ref)

    ws_ref[pl.ds(e, 1), :] += jnp.sum(w_ref[0], axis=-1)[None, :]


def _fin_body(shp_ref, ws_ref, out_ref):
    sh = jnp.sum(shp_ref[...], axis=0)
    out_ref[...] = jnp.sum(sh * ws_ref[...]).reshape(1, 1)


def kernel(hidden_4d, sparsity, gate_up_proj):
    hidden = hidden_4d.reshape(T, H)
    spR = sparsity.reshape(E, NW, TPW).transpose(1, 0, 2)
    w = gate_up_proj.reshape(E, H, O2)

    shp = _sc_sh(spR, hidden)

    ws = pl.pallas_call(
        _ws_body,
        grid=(E, OC),
        in_specs=[pl.BlockSpec((1, H, CH), lambda e, oc: (e, 0, oc))],
        out_specs=pl.BlockSpec((E, H), lambda e, oc: (0, 0)),
        out_shape=jax.ShapeDtypeStruct((E, H), jnp.float32),
    )(w)

    out = pl.pallas_call(
        _fin_body,
        in_specs=[pl.BlockSpec((NW, E, H), lambda: (0, 0, 0)),
                  pl.BlockSpec((E, H), lambda: (0, 0))],
        out_specs=pl.BlockSpec((1, 1), lambda: (0, 0)),
        out_shape=jax.ShapeDtypeStruct((1, 1), jnp.float32),
    )(shp, ws)
    return out[0, 0]
